# baseline (speedup 1.0000x reference)
import jax
import jax.numpy as jnp
from jax import lax
from jax.experimental import pallas as pl
from jax.experimental.pallas import tpu as pltpu
from jax.experimental.pallas import tpu_sc as plsc

IN_DIM = 3840
INTERIM = 960
BLK = 32
NBLOCKS = INTERIM // BLK
FAN = 4
LANES = 16
NC, NS = 2, 16
NW = NC * NS
RCHUNK = 8
NGRP = INTERIM // LANES
ROW_TILE = 512


def _sc_body(x_hbm, g_hbm, out_hbm, g_v, xbuf, ibuf,
             sem_in0, sem_in1, sem_out0, sem_out1):
    n = x_hbm.shape[0]
    rows_per_w = n // NW
    nchunk = rows_per_w // RCHUNK
    wid = lax.axis_index("s") * NC + lax.axis_index("c")
    sem_in = (sem_in0, sem_in1)
    sem_out = (sem_out0, sem_out1)

    pltpu.sync_copy(g_hbm, g_v)

    base_w = wid * rows_per_w
    last = nchunk - 1

    def start_in(c, b):
        pltpu.async_copy(x_hbm.at[pl.ds(base_w + c * RCHUNK, RCHUNK), :],
                         xbuf.at[b], sem_in[b])

    def wait_in(b):
        pltpu.make_async_copy(x_hbm.at[pl.ds(0, RCHUNK), :],
                              xbuf.at[b], sem_in[b]).wait()

    def start_out(c, b):
        pltpu.async_copy(ibuf.at[b], out_hbm.at[pl.ds(base_w + c * RCHUNK, RCHUNK), :],
                         sem_out[b])

    def wait_out(b):
        pltpu.make_async_copy(ibuf.at[b],
                              out_hbm.at[pl.ds(0, RCHUNK), :], sem_out[b]).wait()

    start_in(0, 0)
    start_in(1, 1)

    def compute(c, b):
        xb = xbuf.at[b]

        @plsc.parallel_loop(0, NGRP, unroll=6)
        def grp(j):
            off = j * LANES
            idx = [g_v[t, pl.ds(off, LANES)] for t in range(FAN)]
            for r in range(RCHUNK):
                rv = jnp.full((LANES,), r, jnp.int32)
                a0 = plsc.load_gather(xb, [rv, idx[0]]) + plsc.load_gather(xb, [rv, idx[1]])
                a1 = plsc.load_gather(xb, [rv, idx[2]]) + plsc.load_gather(xb, [rv, idx[3]])
                ibuf[b, r, pl.ds(off, LANES)] = a0 + a1

    def pair(cp, carry):
        for b in range(2):
            c = cp * 2 + b
            wait_in(b)

            @pl.when(cp > 0)
            def _():
                wait_out(b)

            compute(c, b)
            start_out(c, b)
            start_in(jnp.minimum(c + 2, last), b)
        return carry

    lax.fori_loop(0, nchunk // 2, pair, 0)

    for b in range(2):
        wait_in(b)
        wait_out(b)


def _matmul_body(i_ref, bd_ref, o_ref):
    a = i_ref[...].astype(jnp.bfloat16)
    o_ref[...] = jnp.dot(a, bd_ref[...], preferred_element_type=jnp.float32)


def kernel(x, scaling_factors, W, vecin_select_idx, irrep_scatter_idx, interim_l_idx):
    n = x.shape[0]
    vec = vecin_select_idx.astype(jnp.int32)
    scat = irrep_scatter_idx.astype(jnp.int32)
    lidx = interim_l_idx.astype(jnp.int32)

    p = jnp.argsort(scat)
    g = vec[p].reshape(INTERIM, FAN).T

    mesh = plsc.VectorSubcoreMesh(core_axis_name="c", subcore_axis_name="s",
                                  num_cores=NC, num_subcores=NS)
    interim = pl.kernel(
        _sc_body,
        out_type=jax.ShapeDtypeStruct((n, INTERIM), x.dtype),
        mesh=mesh,
        compiler_params=pltpu.CompilerParams(needs_layout_passes=False),
        scratch_types=[
            pltpu.VMEM((FAN, INTERIM), jnp.int32),
            pltpu.VMEM((2, RCHUNK, IN_DIM), jnp.float32),
            pltpu.VMEM((2, RCHUNK, INTERIM), jnp.float32),
            pltpu.SemaphoreType.DMA,
            pltpu.SemaphoreType.DMA,
            pltpu.SemaphoreType.DMA,
            pltpu.SemaphoreType.DMA,
        ],
    )(x, g)

    s_col = scaling_factors[lidx]
    bd = s_col[:, None] * jnp.kron(jnp.eye(NBLOCKS, dtype=x.dtype), W.T)
    bd = bd.astype(jnp.bfloat16)

    out = pl.pallas_call(
        _matmul_body,
        grid=(n // ROW_TILE,),
        in_specs=[
            pl.BlockSpec((ROW_TILE, INTERIM), lambda i: (i, 0)),
            pl.BlockSpec((INTERIM, INTERIM), lambda i: (0, 0)),
        ],
        out_specs=pl.BlockSpec((ROW_TILE, INTERIM), lambda i: (i, 0)),
        out_shape=jax.ShapeDtypeStruct((n, INTERIM), x.dtype),
    )(interim, bd)
    return out


# speedup vs baseline: 1.0260x; 1.0260x over previous
import jax
import jax.numpy as jnp
from jax import lax
from jax.experimental import pallas as pl
from jax.experimental.pallas import tpu as pltpu
from jax.experimental.pallas import tpu_sc as plsc

IN_DIM = 3840
INTERIM = 960
BLK = 32
NBLOCKS = INTERIM // BLK
FAN = 4
LANES = 16
NC, NS = 2, 16
NW = NC * NS
RCHUNK = 8
NGRP = INTERIM // LANES
ROW_TILE = 512


def _sc_body(x_hbm, g_hbm, out_hbm, g_v, xbuf, ibuf,
             sem_in0, sem_in1, sem_out0, sem_out1):
    n = x_hbm.shape[0]
    rows_per_w = n // NW
    nchunk = rows_per_w // RCHUNK
    wid = lax.axis_index("s") * NC + lax.axis_index("c")
    sem_in = (sem_in0, sem_in1)
    sem_out = (sem_out0, sem_out1)

    pltpu.sync_copy(g_hbm, g_v)

    base_w = wid * rows_per_w
    last = nchunk - 1

    def start_in(c, b):
        pltpu.async_copy(x_hbm.at[pl.ds(base_w + c * RCHUNK, RCHUNK), :],
                         xbuf.at[b], sem_in[b])

    def wait_in(b):
        pltpu.make_async_copy(x_hbm.at[pl.ds(0, RCHUNK), :],
                              xbuf.at[b], sem_in[b]).wait()

    def start_out(c, b):
        pltpu.async_copy(ibuf.at[b], out_hbm.at[pl.ds(base_w + c * RCHUNK, RCHUNK), :],
                         sem_out[b])

    def wait_out(b):
        pltpu.make_async_copy(ibuf.at[b],
                              out_hbm.at[pl.ds(0, RCHUNK), :], sem_out[b]).wait()

    start_in(0, 0)
    start_in(1, 1)

    def compute(c, b):
        xb = xbuf.at[b]

        @plsc.parallel_loop(0, NGRP, unroll=4)
        def grp(j):
            off = j * LANES
            idx = [g_v[t, pl.ds(off, LANES)] for t in range(FAN)]
            for r in range(RCHUNK):
                rv = jnp.full((LANES,), r, jnp.int32)
                a0 = plsc.load_gather(xb, [rv, idx[0]]) + plsc.load_gather(xb, [rv, idx[1]])
                a1 = plsc.load_gather(xb, [rv, idx[2]]) + plsc.load_gather(xb, [rv, idx[3]])
                ibuf[b, r, pl.ds(off, LANES)] = a0 + a1

    def pair(cp, carry):
        for b in range(2):
            c = cp * 2 + b
            wait_in(b)

            @pl.when(cp > 0)
            def _():
                wait_out(b)

            compute(c, b)
            start_out(c, b)
            start_in(jnp.minimum(c + 2, last), b)
        return carry

    lax.fori_loop(0, nchunk // 2, pair, 0)

    for b in range(2):
        wait_in(b)
        wait_out(b)


def _matmul_body(i_ref, bd_ref, o_ref):
    a = i_ref[...].astype(jnp.bfloat16)
    o_ref[...] = jnp.dot(a, bd_ref[...], preferred_element_type=jnp.float32)


def kernel(x, scaling_factors, W, vecin_select_idx, irrep_scatter_idx, interim_l_idx):
    n = x.shape[0]
    vec = vecin_select_idx.astype(jnp.int32)
    scat = irrep_scatter_idx.astype(jnp.int32)
    lidx = interim_l_idx.astype(jnp.int32)

    p = jnp.argsort(scat)
    g = vec[p].reshape(INTERIM, FAN).T

    mesh = plsc.VectorSubcoreMesh(core_axis_name="c", subcore_axis_name="s",
                                  num_cores=NC, num_subcores=NS)
    interim = pl.kernel(
        _sc_body,
        out_type=jax.ShapeDtypeStruct((n, INTERIM), x.dtype),
        mesh=mesh,
        compiler_params=pltpu.CompilerParams(needs_layout_passes=False, skip_device_barrier=True),
        scratch_types=[
            pltpu.VMEM((FAN, INTERIM), jnp.int32),
            pltpu.VMEM((2, RCHUNK, IN_DIM), jnp.float32),
            pltpu.VMEM((2, RCHUNK, INTERIM), jnp.float32),
            pltpu.SemaphoreType.DMA,
            pltpu.SemaphoreType.DMA,
            pltpu.SemaphoreType.DMA,
            pltpu.SemaphoreType.DMA,
        ],
    )(x, g)

    s_col = scaling_factors[lidx]
    bd = s_col[:, None] * jnp.kron(jnp.eye(NBLOCKS, dtype=x.dtype), W.T)
    bd = bd.astype(jnp.bfloat16)

    out = pl.pallas_call(
        _matmul_body,
        grid=(n // ROW_TILE,),
        in_specs=[
            pl.BlockSpec((ROW_TILE, INTERIM), lambda i: (i, 0)),
            pl.BlockSpec((INTERIM, INTERIM), lambda i: (0, 0)),
        ],
        out_specs=pl.BlockSpec((ROW_TILE, INTERIM), lambda i: (i, 0)),
        out_shape=jax.ShapeDtypeStruct((n, INTERIM), x.dtype),
    )(interim, bd)
    return out


# speedup vs baseline: 1.0593x; 1.0325x over previous
import jax
import jax.numpy as jnp
from jax import lax
from jax.experimental import pallas as pl
from jax.experimental.pallas import tpu as pltpu
from jax.experimental.pallas import tpu_sc as plsc

IN_DIM = 3840
INTERIM = 960
BLK = 32
NBLOCKS = INTERIM // BLK
FAN = 4
LANES = 16
NC, NS = 2, 16
NW = NC * NS
RCHUNK = 8
NGRP = INTERIM // LANES
ROW_TILE = 1024


def _sc_body(x_hbm, g_hbm, out_hbm, g_v, xbuf, ibuf,
             sem_in0, sem_in1, sem_out0, sem_out1):
    n = x_hbm.shape[0]
    rows_per_w = n // NW
    nchunk = rows_per_w // RCHUNK
    wid = lax.axis_index("s") * NC + lax.axis_index("c")
    sem_in = (sem_in0, sem_in1)
    sem_out = (sem_out0, sem_out1)

    pltpu.sync_copy(g_hbm, g_v)

    base_w = wid * rows_per_w
    last = nchunk - 1

    def start_in(c, b):
        pltpu.async_copy(x_hbm.at[pl.ds(base_w + c * RCHUNK, RCHUNK), :],
                         xbuf.at[b], sem_in[b])

    def wait_in(b):
        pltpu.make_async_copy(x_hbm.at[pl.ds(0, RCHUNK), :],
                              xbuf.at[b], sem_in[b]).wait()

    def start_out(c, b):
        pltpu.async_copy(ibuf.at[b], out_hbm.at[pl.ds(base_w + c * RCHUNK, RCHUNK), :],
                         sem_out[b])

    def wait_out(b):
        pltpu.make_async_copy(ibuf.at[b],
                              out_hbm.at[pl.ds(0, RCHUNK), :], sem_out[b]).wait()

    start_in(0, 0)
    start_in(1, 1)

    def compute(c, b):
        xb = xbuf.at[b]

        @plsc.parallel_loop(0, NGRP, unroll=4)
        def grp(j):
            off = j * LANES
            idx = [g_v[t, pl.ds(off, LANES)] for t in range(FAN)]
            for r in range(RCHUNK):
                rv = jnp.full((LANES,), r, jnp.int32)
                a0 = plsc.load_gather(xb, [rv, idx[0]]) + plsc.load_gather(xb, [rv, idx[1]])
                a1 = plsc.load_gather(xb, [rv, idx[2]]) + plsc.load_gather(xb, [rv, idx[3]])
                ibuf[b, r, pl.ds(off, LANES)] = a0 + a1

    def pair(cp, carry):
        for b in range(2):
            c = cp * 2 + b
            wait_in(b)

            @pl.when(cp > 0)
            def _():
                wait_out(b)

            compute(c, b)
            start_out(c, b)
            start_in(jnp.minimum(c + 2, last), b)
        return carry

    lax.fori_loop(0, nchunk // 2, pair, 0)

    for b in range(2):
        wait_in(b)
        wait_out(b)


def _matmul_body(i_ref, bd_ref, o_ref):
    a = i_ref[...].astype(jnp.bfloat16)
    o_ref[...] = jnp.dot(a, bd_ref[...], preferred_element_type=jnp.float32)


def kernel(x, scaling_factors, W, vecin_select_idx, irrep_scatter_idx, interim_l_idx):
    n = x.shape[0]
    vec = vecin_select_idx.astype(jnp.int32)
    scat = irrep_scatter_idx.astype(jnp.int32)
    lidx = interim_l_idx.astype(jnp.int32)

    p = jnp.argsort(scat)
    g = vec[p].reshape(INTERIM, FAN).T

    mesh = plsc.VectorSubcoreMesh(core_axis_name="c", subcore_axis_name="s",
                                  num_cores=NC, num_subcores=NS)
    interim = pl.kernel(
        _sc_body,
        out_type=jax.ShapeDtypeStruct((n, INTERIM), x.dtype),
        mesh=mesh,
        compiler_params=pltpu.CompilerParams(needs_layout_passes=False),
        scratch_types=[
            pltpu.VMEM((FAN, INTERIM), jnp.int32),
            pltpu.VMEM((2, RCHUNK, IN_DIM), jnp.float32),
            pltpu.VMEM((2, RCHUNK, INTERIM), jnp.float32),
            pltpu.SemaphoreType.DMA,
            pltpu.SemaphoreType.DMA,
            pltpu.SemaphoreType.DMA,
            pltpu.SemaphoreType.DMA,
        ],
    )(x, g)

    s_col = scaling_factors[lidx]
    bd = s_col[:, None] * jnp.kron(jnp.eye(NBLOCKS, dtype=x.dtype), W.T)
    bd = bd.astype(jnp.bfloat16)

    out = pl.pallas_call(
        _matmul_body,
        grid=(n // ROW_TILE,),
        in_specs=[
            pl.BlockSpec((ROW_TILE, INTERIM), lambda i: (i, 0)),
            pl.BlockSpec((INTERIM, INTERIM), lambda i: (0, 0)),
        ],
        out_specs=pl.BlockSpec((ROW_TILE, INTERIM), lambda i: (i, 0)),
        out_shape=jax.ShapeDtypeStruct((n, INTERIM), x.dtype),
    )(interim, bd)
    return out


# speedup vs baseline: 1.0705x; 1.0105x over previous
import jax
import jax.numpy as jnp
from jax import lax
from jax.experimental import pallas as pl
from jax.experimental.pallas import tpu as pltpu
from jax.experimental.pallas import tpu_sc as plsc

IN_DIM = 3840
INTERIM = 960
BLK = 32
NBLOCKS = INTERIM // BLK
FAN = 4
LANES = 16
NC, NS = 2, 16
NW = NC * NS
RCHUNK = 8
NGRP = INTERIM // LANES
ROW_TILE = 2048


def _sc_body(x_hbm, g_hbm, out_hbm, g_v, xbuf, ibuf,
             sem_in0, sem_in1, sem_out0, sem_out1):
    n = x_hbm.shape[0]
    rows_per_w = n // NW
    nchunk = rows_per_w // RCHUNK
    wid = lax.axis_index("s") * NC + lax.axis_index("c")
    sem_in = (sem_in0, sem_in1)
    sem_out = (sem_out0, sem_out1)

    pltpu.sync_copy(g_hbm, g_v)

    base_w = wid * rows_per_w
    last = nchunk - 1

    def start_in(c, b):
        pltpu.async_copy(x_hbm.at[pl.ds(base_w + c * RCHUNK, RCHUNK), :],
                         xbuf.at[b], sem_in[b])

    def wait_in(b):
        pltpu.make_async_copy(x_hbm.at[pl.ds(0, RCHUNK), :],
                              xbuf.at[b], sem_in[b]).wait()

    def start_out(c, b):
        pltpu.async_copy(ibuf.at[b], out_hbm.at[pl.ds(base_w + c * RCHUNK, RCHUNK), :],
                         sem_out[b])

    def wait_out(b):
        pltpu.make_async_copy(ibuf.at[b],
                              out_hbm.at[pl.ds(0, RCHUNK), :], sem_out[b]).wait()

    start_in(0, 0)
    start_in(1, 1)

    def compute(c, b):
        xb = xbuf.at[b]

        @plsc.parallel_loop(0, NGRP, unroll=4)
        def grp(j):
            off = j * LANES
            idx = [g_v[t, pl.ds(off, LANES)] for t in range(FAN)]
            for r in range(RCHUNK):
                rv = jnp.full((LANES,), r, jnp.int32)
                a0 = plsc.load_gather(xb, [rv, idx[0]]) + plsc.load_gather(xb, [rv, idx[1]])
                a1 = plsc.load_gather(xb, [rv, idx[2]]) + plsc.load_gather(xb, [rv, idx[3]])
                ibuf[b, r, pl.ds(off, LANES)] = a0 + a1

    def pair(cp, carry):
        for b in range(2):
            c = cp * 2 + b
            wait_in(b)

            @pl.when(cp > 0)
            def _():
                wait_out(b)

            compute(c, b)
            start_out(c, b)
            start_in(jnp.minimum(c + 2, last), b)
        return carry

    lax.fori_loop(0, nchunk // 2, pair, 0)

    for b in range(2):
        wait_in(b)
        wait_out(b)


def _matmul_body(i_ref, bd_ref, o_ref):
    a = i_ref[...].astype(jnp.bfloat16)
    o_ref[...] = jnp.dot(a, bd_ref[...], preferred_element_type=jnp.float32)


def kernel(x, scaling_factors, W, vecin_select_idx, irrep_scatter_idx, interim_l_idx):
    n = x.shape[0]
    vec = vecin_select_idx.astype(jnp.int32)
    scat = irrep_scatter_idx.astype(jnp.int32)
    lidx = interim_l_idx.astype(jnp.int32)

    p = jnp.argsort(scat)
    g = vec[p].reshape(INTERIM, FAN).T

    mesh = plsc.VectorSubcoreMesh(core_axis_name="c", subcore_axis_name="s",
                                  num_cores=NC, num_subcores=NS)
    interim = pl.kernel(
        _sc_body,
        out_type=jax.ShapeDtypeStruct((n, INTERIM), x.dtype),
        mesh=mesh,
        compiler_params=pltpu.CompilerParams(needs_layout_passes=False),
        scratch_types=[
            pltpu.VMEM((FAN, INTERIM), jnp.int32),
            pltpu.VMEM((2, RCHUNK, IN_DIM), jnp.float32),
            pltpu.VMEM((2, RCHUNK, INTERIM), jnp.float32),
            pltpu.SemaphoreType.DMA,
            pltpu.SemaphoreType.DMA,
            pltpu.SemaphoreType.DMA,
            pltpu.SemaphoreType.DMA,
        ],
    )(x, g)

    s_col = scaling_factors[lidx]
    bd = s_col[:, None] * jnp.kron(jnp.eye(NBLOCKS, dtype=x.dtype), W.T)
    bd = bd.astype(jnp.bfloat16)

    out = pl.pallas_call(
        _matmul_body,
        grid=(n // ROW_TILE,),
        in_specs=[
            pl.BlockSpec((ROW_TILE, INTERIM), lambda i: (i, 0)),
            pl.BlockSpec((INTERIM, INTERIM), lambda i: (0, 0)),
        ],
        out_specs=pl.BlockSpec((ROW_TILE, INTERIM), lambda i: (i, 0)),
        out_shape=jax.ShapeDtypeStruct((n, INTERIM), x.dtype),
    )(interim, bd)
    return out
